# baseline (speedup 1.0000x reference)
import functools

import jax
import jax.numpy as jnp
import numpy as np
from jax import lax
from jax.experimental import pallas as pl
from jax.experimental.pallas import tpu as pltpu
from jax.experimental.pallas import tpu_sc as plsc

_PARENTS = np.array([-1, 0, 1, 2, 3, 1, 5, 6, 1, 8, 9, 10, 8, 12, 13],
                    dtype=np.int32)

_B, _T, _J, _C = 4096, 200, 15, 3
_NWORKERS = 32
_BANDS = _T // 8
_COLG = _B // 128
_NTASKS = _BANDS * _COLG * _C
_TASKS_PER_W = _NTASKS // _NWORKERS
_NRING = 4


def _ik_body(y_hbm, out_hbm, b0, b1, b2, b3,
             si0, si1, si2, si3, so0, so1, so2, so3):
    bufs = (b0, b1, b2, b3)
    sins = (si0, si1, si2, si3)
    souts = (so0, so1, so2, so3)

    cid = lax.axis_index("c")
    sid = lax.axis_index("s")
    wid = sid * 2 + cid
    t0 = wid * _TASKS_PER_W

    def unit_slices(t):
        tid = t0 + t
        band = tid // (_COLG * _C)
        rem = tid % (_COLG * _C)
        colg = rem // _C
        c = rem % _C
        return c, pl.ds(band * 8, 8), pl.ds(colg * 128, 128)

    def in_copy(b, t):
        c, rs, cs = unit_slices(t)
        return pltpu.make_async_copy(
            y_hbm.at[:, c, rs, cs], bufs[b], sins[b])

    def out_copy(b, t):
        c, rs, cs = unit_slices(t)
        return pltpu.make_async_copy(
            bufs[b], out_hbm.at[:, c, rs, cs], souts[b])

    def compute(b):
        buf = bufs[b]

        @pl.loop(0, 8)
        def _row(r):
            for l in range(8):
                sl = pl.ds(l * 16, 16)
                v = [None] * _J
                v[0] = buf[0, r, sl]
                for j in range(1, _J):
                    v[j] = buf[j, r, sl]
                    buf[j, r, sl] = v[j] - v[int(_PARENTS[j])]

    in_copy(0, 0).start()
    in_copy(1, 1).start()
    for t in range(4):
        b = t % _NRING
        br = (t + 2) % _NRING
        in_copy(b, t).wait()
        compute(b)
        out_copy(b, t).start()
        if t >= 2:
            out_copy(br, t - 2).wait()
        in_copy(br, t + 2).start()

    @pl.loop(1, (_TASKS_PER_W - 4) // _NRING + 1)
    def _quad(g):
        for k in range(_NRING):
            t = g * _NRING + k
            b = k
            br = (k + 2) % _NRING
            in_copy(b, t).wait()
            compute(b)
            out_copy(b, t).start()
            out_copy(br, t - 2).wait()
            in_copy(br, t + 2).start()

    for t in range(_TASKS_PER_W - 3, _TASKS_PER_W):
        b = t % _NRING
        br = (t + 2) % _NRING
        in_copy(b, t).wait()
        compute(b)
        out_copy(b, t).start()
        if t + 2 < _TASKS_PER_W:
            out_copy(br, t - 2).wait()
            in_copy(br, t + 2).start()

    for t in range(_TASKS_PER_W - 4, _TASKS_PER_W):
        out_copy(t % _NRING, t).wait()


@jax.jit
def _ik_planes(y):
    mesh = plsc.VectorSubcoreMesh(core_axis_name="c", subcore_axis_name="s")
    return pl.kernel(
        _ik_body,
        out_type=jax.ShapeDtypeStruct((_J, _C, _T, _B), jnp.float32),
        mesh=mesh,
        scratch_types=(
            [pltpu.VMEM((_J, 8, 128), jnp.float32) for _ in range(_NRING)]
            + [pltpu.SemaphoreType.DMA] * 8),
        compiler_params=pltpu.CompilerParams(
            needs_layout_passes=False, use_tc_tiling_on_sc=True),
    )(y)


def kernel(x):
    y = jnp.transpose(x, (2, 3, 1, 0))
    out = _ik_planes(y)
    return jnp.transpose(out, (3, 2, 0, 1))


# speedup vs baseline: 1.0007x; 1.0007x over previous
import functools

import jax
import jax.numpy as jnp
import numpy as np
from jax import lax
from jax.experimental import pallas as pl
from jax.experimental.pallas import tpu as pltpu
from jax.experimental.pallas import tpu_sc as plsc

_PARENTS = np.array([-1, 0, 1, 2, 3, 1, 5, 6, 1, 8, 9, 10, 8, 12, 13],
                    dtype=np.int32)

_B, _T, _J, _C = 4096, 200, 15, 3
_NWORKERS = 32
_BANDS = _T // 8
_COLG = _B // 128
_NTASKS = _BANDS * _COLG * _C
_TASKS_PER_W = _NTASKS // _NWORKERS
_NRING = 6
_DIST = _NRING // 2


def _ik_body(y_hbm, out_hbm, b0, b1, b2, b3, b4, b5,
             si0, si1, si2, si3, si4, si5,
             so0, so1, so2, so3, so4, so5):
    bufs = (b0, b1, b2, b3, b4, b5)
    sins = (si0, si1, si2, si3, si4, si5)
    souts = (so0, so1, so2, so3, so4, so5)

    cid = lax.axis_index("c")
    sid = lax.axis_index("s")
    wid = sid * 2 + cid
    t0 = wid * _TASKS_PER_W

    def unit_slices(t):
        tid = t0 + t
        band = tid // (_COLG * _C)
        rem = tid % (_COLG * _C)
        colg = rem // _C
        c = rem % _C
        return c, pl.ds(band * 8, 8), pl.ds(colg * 128, 128)

    def in_copy(b, t):
        c, rs, cs = unit_slices(t)
        return pltpu.make_async_copy(
            y_hbm.at[:, c, rs, cs], bufs[b], sins[b])

    def out_copy(b, t):
        c, rs, cs = unit_slices(t)
        return pltpu.make_async_copy(
            bufs[b], out_hbm.at[:, c, rs, cs], souts[b])

    def compute(b):
        buf = bufs[b]

        @pl.loop(0, 8)
        def _row(r):
            for l in range(8):
                sl = pl.ds(l * 16, 16)
                v = [None] * _J
                v[0] = buf[0, r, sl]
                for j in range(1, _J):
                    v[j] = buf[j, r, sl]
                    buf[j, r, sl] = v[j] - v[int(_PARENTS[j])]

    for t in range(_DIST):
        in_copy(t, t).start()
    for t in range(_NRING):
        b = t % _NRING
        br = (t + _DIST) % _NRING
        in_copy(b, t).wait()
        compute(b)
        out_copy(b, t).start()
        if t >= _DIST:
            out_copy(br, t - _DIST).wait()
        in_copy(br, t + _DIST).start()

    @pl.loop(1, (_TASKS_PER_W - _NRING + _DIST) // _NRING)
    def _ring(g):
        for k in range(_NRING):
            t = g * _NRING + k
            b = k
            br = (k + _DIST) % _NRING
            in_copy(b, t).wait()
            compute(b)
            out_copy(b, t).start()
            out_copy(br, t - _DIST).wait()
            in_copy(br, t + _DIST).start()

    for t in range(_TASKS_PER_W - _DIST, _TASKS_PER_W):
        b = t % _NRING
        br = (t + _DIST) % _NRING
        in_copy(b, t).wait()
        compute(b)
        out_copy(b, t).start()
        if t + _DIST < _TASKS_PER_W:
            out_copy(br, t - _DIST).wait()
            in_copy(br, t + _DIST).start()

    for t in range(_TASKS_PER_W - 2 * _DIST, _TASKS_PER_W):
        out_copy(t % _NRING, t).wait()


@jax.jit
def _ik_planes(y):
    mesh = plsc.VectorSubcoreMesh(core_axis_name="c", subcore_axis_name="s")
    return pl.kernel(
        _ik_body,
        out_type=jax.ShapeDtypeStruct((_J, _C, _T, _B), jnp.float32),
        mesh=mesh,
        scratch_types=(
            [pltpu.VMEM((_J, 8, 128), jnp.float32) for _ in range(_NRING)]
            + [pltpu.SemaphoreType.DMA] * (2 * _NRING)),
        compiler_params=pltpu.CompilerParams(
            needs_layout_passes=False, use_tc_tiling_on_sc=True),
    )(y)


def kernel(x):
    y = jnp.transpose(x, (2, 3, 1, 0))
    out = _ik_planes(y)
    return jnp.transpose(out, (3, 2, 0, 1))
